# baseline (speedup 1.0000x reference)
import jax
import jax.numpy as jnp
from jax import lax
from jax.experimental import pallas as pl
from jax.experimental.pallas import tpu as pltpu
from jax.experimental.pallas import tpu_sc as plsc

N_CHANNELS = 4096
N_BINS = 8192
LANES = 16
N_WORKERS = 32
SC_ROWS = 1024
TC_ROWS = N_CHANNELS - SC_ROWS
ROWS_PER_WORKER = SC_ROWS // N_WORKERS
RB = 4
N_BLOCKS = ROWS_PER_WORKER // RB
UNROLL = 8
N_CHUNKS = N_BINS // LANES
BLOCK_R = 256


def _flux_body(params_ref, elow_ref, ehigh_ref):
    norm = params_ref[0]
    alpha = params_ref[1]
    e_low = elow_ref[...]
    e_high = ehigh_ref[...]
    e_mid = 0.5 * (e_low + e_high)
    de = e_high - e_low
    return norm * jnp.exp(-alpha * jnp.log(e_mid)) * de


def _flux_kernel(params_ref, elow_ref, ehigh_ref, flux_ref):
    flux_ref[...] = _flux_body(params_ref, elow_ref, ehigh_ref)


def _compute_flux(parameters, e_low, e_high):
    out = pl.pallas_call(
        _flux_kernel,
        in_specs=[
            pl.BlockSpec(memory_space=pltpu.SMEM),
            pl.BlockSpec((1, N_BINS), lambda: (0, 0)),
            pl.BlockSpec((1, N_BINS), lambda: (0, 0)),
        ],
        out_specs=pl.BlockSpec((1, N_BINS), lambda: (0, 0)),
        out_shape=jax.ShapeDtypeStruct((1, N_BINS), jnp.float32),
    )(parameters, e_low.reshape(1, N_BINS), e_high.reshape(1, N_BINS))
    return out.reshape(N_BINS)


def _tc_mv_kernel(params_ref, elow_ref, ehigh_ref, tm_ref, out_ref):
    flux = _flux_body(params_ref, elow_ref, ehigh_ref)
    acc = jax.lax.dot_general(
        tm_ref[...], flux,
        dimension_numbers=(((1,), (1,)), ((), ())),
        preferred_element_type=jnp.float32,
    )
    out_ref[...] = jnp.maximum(acc, 1e-6)


def _tc_matvec(parameters, transfer_matrix, e_low, e_high):
    out = pl.pallas_call(
        _tc_mv_kernel,
        grid=(TC_ROWS // BLOCK_R,),
        in_specs=[
            pl.BlockSpec(memory_space=pltpu.SMEM),
            pl.BlockSpec((1, N_BINS), lambda i: (0, 0)),
            pl.BlockSpec((1, N_BINS), lambda i: (0, 0)),
            pl.BlockSpec((BLOCK_R, N_BINS), lambda i: (i, 0)),
        ],
        out_specs=pl.BlockSpec((BLOCK_R, 1), lambda i: (i, 0)),
        out_shape=jax.ShapeDtypeStruct((TC_ROWS, 1), jnp.float32),
    )(parameters, e_low.reshape(1, N_BINS), e_high.reshape(1, N_BINS),
      transfer_matrix)
    return out.reshape(TC_ROWS)


def _sc_matvec_body(tm_hbm, flux_hbm, out_hbm, flux_v, buf_v, out_v, sems):
    wid = lax.axis_index("s") * 2 + lax.axis_index("c")
    row0 = TC_ROWS + wid * ROWS_PER_WORKER
    pltpu.sync_copy(flux_hbm.at[:], flux_v)
    lane_iota = lax.iota(jnp.int32, LANES)
    zero_vec = lax.convert_element_type(lane_iota & 0, jnp.float32)

    def start_fetch(b, slot):
        return pltpu.async_copy(
            tm_hbm.at[pl.ds(row0 + b * RB, RB), :], buf_v.at[slot],
            sems.at[slot])

    copies = [None] * N_BLOCKS
    copies[0] = start_fetch(0, 0)
    res = zero_vec
    for b in range(N_BLOCKS):
        slot = b % 2
        if b + 1 < N_BLOCKS:
            copies[b + 1] = start_fetch(b + 1, 1 - slot)
        copies[b].wait()

        def chunk_body(i, accs, _slot=slot):
            base = i * (LANES * UNROLL)
            for u in range(UNROLL):
                off = base + u * LANES
                f = flux_v[pl.ds(off, LANES)]
                accs = tuple(
                    accs[r] + buf_v[_slot, r, pl.ds(off, LANES)] * f
                    for r in range(RB)
                )
            return accs

        zeros = tuple(zero_vec for _ in range(RB))
        accs = lax.fori_loop(0, N_CHUNKS // UNROLL, chunk_body, zeros)
        for r in range(RB):
            v = accs[r]
            for sh in (8, 4, 2, 1):
                v = v + _lane_shuffle(v, lane_iota ^ sh)
            pos = (b * RB + r) % LANES
            res = jnp.where(lane_iota == pos, v, res)
        if (b + 1) % (LANES // RB) == 0:
            vec = jnp.maximum(res, jnp.float32(1e-6))
            g = (b + 1) * RB - LANES
            out_v[pl.ds(g, LANES)] = vec
            res = zero_vec

    pltpu.sync_copy(out_v, out_hbm.at[pl.ds(wid * ROWS_PER_WORKER,
                                            ROWS_PER_WORKER)])


def _lane_shuffle(v, perm):
    return lax.gather(
        v, perm[:, None],
        dimension_numbers=lax.GatherDimensionNumbers(
            offset_dims=(), collapsed_slice_dims=(0,), start_index_map=(0,)),
        slice_sizes=(1,),
        mode=lax.GatherScatterMode.PROMISE_IN_BOUNDS,
    )


def _sc_matvec(transfer_matrix, flux):
    mesh = plsc.VectorSubcoreMesh(core_axis_name="c", subcore_axis_name="s")
    sc = pl.kernel(
        _sc_matvec_body,
        mesh=mesh,
        out_type=jax.ShapeDtypeStruct((SC_ROWS,), jnp.float32),
        scratch_types=[
            pltpu.VMEM((N_BINS,), jnp.float32),
            pltpu.VMEM((2, RB, N_BINS), jnp.float32),
            pltpu.VMEM((ROWS_PER_WORKER,), jnp.float32),
            pltpu.SemaphoreType.DMA((2,)),
        ],
    )
    return sc(transfer_matrix, flux)


def kernel(parameters, transfer_matrix, e_low, e_high):
    flux = _compute_flux(parameters, e_low, e_high)
    out_tc = _tc_matvec(parameters, transfer_matrix, e_low, e_high)
    out_sc = _sc_matvec(transfer_matrix, flux)
    return jnp.concatenate([out_tc, out_sc])


# speedup vs baseline: 1.2715x; 1.2715x over previous
import jax
import jax.numpy as jnp
from jax.experimental import pallas as pl
from jax.experimental.pallas import tpu as pltpu

N_CHANNELS = 4096
N_BINS = 8192
BLOCK_R = 128


def _mv_kernel(params_ref, elow_ref, ehigh_ref, tm_ref, out_ref):
    norm = params_ref[0]
    alpha = params_ref[1]
    e_low = elow_ref[...]
    e_high = ehigh_ref[...]
    e_mid = 0.5 * (e_low + e_high)
    de = e_high - e_low
    flux = norm * jnp.exp(-alpha * jnp.log(e_mid)) * de
    acc = jax.lax.dot_general(
        tm_ref[...], flux,
        dimension_numbers=(((1,), (1,)), ((), ())),
        preferred_element_type=jnp.float32,
    )
    out_ref[...] = jnp.maximum(acc, 1e-6)


def kernel(parameters, transfer_matrix, e_low, e_high):
    e_low2 = e_low.reshape(1, N_BINS)
    e_high2 = e_high.reshape(1, N_BINS)
    out = pl.pallas_call(
        _mv_kernel,
        grid=(N_CHANNELS // BLOCK_R,),
        in_specs=[
            pl.BlockSpec(memory_space=pltpu.SMEM),
            pl.BlockSpec((1, N_BINS), lambda i: (0, 0)),
            pl.BlockSpec((1, N_BINS), lambda i: (0, 0)),
            pl.BlockSpec((BLOCK_R, N_BINS), lambda i: (i, 0)),
        ],
        out_specs=pl.BlockSpec((BLOCK_R, 1), lambda i: (i, 0)),
        out_shape=jax.ShapeDtypeStruct((N_CHANNELS, 1), jnp.float32),
    )(parameters, e_low2, e_high2, transfer_matrix)
    return out.reshape(N_CHANNELS)


# speedup vs baseline: 1.3459x; 1.0586x over previous
import jax
import jax.numpy as jnp
from jax.experimental import pallas as pl
from jax.experimental.pallas import tpu as pltpu

N_CHANNELS = 4096
N_BINS = 8192
BLOCK_R = 512


def _mv_kernel(params_ref, elow_ref, ehigh_ref, tm_ref, out_ref):
    norm = params_ref[0]
    alpha = params_ref[1]
    e_low = elow_ref[...]
    e_high = ehigh_ref[...]
    e_mid = 0.5 * (e_low + e_high)
    de = e_high - e_low
    flux = norm * jnp.exp(-alpha * jnp.log(e_mid)) * de
    acc = jax.lax.dot_general(
        tm_ref[...], flux,
        dimension_numbers=(((1,), (1,)), ((), ())),
        preferred_element_type=jnp.float32,
    )
    out_ref[...] = jnp.maximum(acc, 1e-6)


def kernel(parameters, transfer_matrix, e_low, e_high):
    e_low2 = e_low.reshape(1, N_BINS)
    e_high2 = e_high.reshape(1, N_BINS)
    out = pl.pallas_call(
        _mv_kernel,
        grid=(N_CHANNELS // BLOCK_R,),
        in_specs=[
            pl.BlockSpec(memory_space=pltpu.SMEM),
            pl.BlockSpec((1, N_BINS), lambda i: (0, 0)),
            pl.BlockSpec((1, N_BINS), lambda i: (0, 0)),
            pl.BlockSpec((BLOCK_R, N_BINS), lambda i: (i, 0)),
        ],
        out_specs=pl.BlockSpec((BLOCK_R, 1), lambda i: (i, 0)),
        out_shape=jax.ShapeDtypeStruct((N_CHANNELS, 1), jnp.float32),
    )(parameters, e_low2, e_high2, transfer_matrix)
    return out.reshape(N_CHANNELS)
